# baseline (speedup 1.0000x reference)
import functools

import jax
import jax.numpy as jnp
from jax import lax
from jax.experimental import pallas as pl
from jax.experimental.pallas import tpu as pltpu
from jax.experimental.pallas import tpu_sc as plsc

N_NODES = 10000
D_FEAT = 128
N_EDGES = 320000

NC = 2
NS = 16
NW = NC * NS
LANES = 16

EPW = N_EDGES // NW
CHUNK = 32
NCHUNK = 312
TAIL = EPW - NCHUNK * CHUNK
GROUPS = CHUNK // LANES
NBUF = 4


def _sc_body(table, nids32, score_out, srch_out, tgth_out,
             score_all, table_sh,
             rows_s0, rows_t0, rows_s1, rows_t1,
             rows_s2, rows_t2, rows_s3, rows_t3,
             ixs0, ixt0, ixs1, ixt1, ixs2, ixt2, ixs3, ixt3,
             trows_s, trows_t, tixs, tixt, fold_v,
             g0, g1, g2, g3, w0, w1, w2, w3, i0, i1, i2, i3):
  sid = lax.axis_index("s")
  wid = sid * NC + lax.axis_index("c")
  base = wid * EPW

  stripe = 632

  @pl.when(sid < NS - 1)
  def _():
    roff = pl.multiple_of(sid * stripe, 8)
    pltpu.sync_copy(table.at[pl.ds(roff, stripe)],
                    table_sh.at[pl.ds(roff, stripe)])

  @pl.when(sid == NS - 1)
  def _():
    tail = N_NODES - (NS - 1) * stripe
    pltpu.sync_copy(table.at[pl.ds((NS - 1) * stripe, tail)],
                    table_sh.at[pl.ds((NS - 1) * stripe, tail)])

  plsc.subcore_barrier()

  rows_s = (rows_s0, rows_s1, rows_s2, rows_s3)
  rows_t = (rows_t0, rows_t1, rows_t2, rows_t3)
  idx_s = (ixs0, ixs1, ixs2, ixs3)
  idx_t = (ixt0, ixt1, ixt2, ixt3)
  gsem = (g0, g1, g2, g3)
  wsem = (w0, w1, w2, w3)
  isem = (i0, i1, i2, i3)

  lane = lax.iota(jnp.int32, LANES)
  fold_v[pl.ds(LANES, LANES)] = jnp.zeros((LANES,), jnp.float32)

  def fire_idx(c, s):
    ioff = pl.multiple_of(base + c * CHUNK, 8)
    pltpu.async_copy(nids32.at[pl.ds(ioff, CHUNK)], idx_s[s], isem[s])
    pltpu.async_copy(nids32.at[pl.ds(N_EDGES + ioff, CHUNK)],
                     idx_t[s], isem[s])

  def drain_idx(s):
    pltpu.make_async_copy(nids32.at[pl.ds(0, CHUNK)], idx_s[s], isem[s]).wait()
    pltpu.make_async_copy(nids32.at[pl.ds(0, CHUNK)], idx_t[s], isem[s]).wait()

  def fire(s):
    pltpu.async_copy(table_sh.at[idx_s[s]], rows_s[s], gsem[s])
    pltpu.async_copy(table_sh.at[idx_t[s]], rows_t[s], gsem[s])

  def drain_gather(s):
    pltpu.make_async_copy(table.at[pl.ds(0, CHUNK)], rows_s[s], gsem[s]).wait()
    pltpu.make_async_copy(table.at[pl.ds(0, CHUNK)], rows_t[s], gsem[s]).wait()

  def drain_wb(s):
    pltpu.make_async_copy(rows_s[s], srch_out.at[pl.ds(0, CHUNK)],
                          wsem[s]).wait()
    pltpu.make_async_copy(rows_t[s], tgth_out.at[pl.ds(0, CHUNK)],
                          wsem[s]).wait()

  def dot_group(rs, rt, g, sbase):
    svec = jnp.zeros((LANES,), jnp.float32)
    for j in range(LANES):
      e = g * LANES + j
      acc = jnp.zeros((LANES,), jnp.float32)
      for k in range(D_FEAT // LANES):
        a = rs[e, pl.ds(k * LANES, LANES)]
        b = rt[e, pl.ds(k * LANES, LANES)]
        acc = acc + a * b
      fold_v[pl.ds(0, LANES)] = acc
      acc = acc + fold_v[pl.ds(LANES // 2, LANES)]
      tot = acc[0]
      for l in range(1, LANES // 2):
        tot = tot + acc[l]
      svec = jnp.where(lane == j, tot, svec)
    score_all[pl.ds(sbase + g * LANES, LANES)] = svec

  def compute(c, s):
    rs, rt = rows_s[s], rows_t[s]
    sbase = c * CHUNK
    for g in range(GROUPS):
      dot_group(rs, rt, g, sbase)

  def issue_wb(c, s):
    off = pl.multiple_of(base + c * CHUNK, 8)
    pltpu.async_copy(rows_s[s], srch_out.at[pl.ds(off, CHUNK)], wsem[s])
    pltpu.async_copy(rows_t[s], tgth_out.at[pl.ds(off, CHUNK)], wsem[s])

  fire_idx(0, 0)
  fire_idx(1, 1)
  fire_idx(2, 2)
  fire_idx(3, 3)
  drain_idx(0)
  fire(0)
  drain_idx(1)
  fire(1)

  def body(p, carry):
    for u in range(NBUF):
      c = p * NBUF + u
      s = u
      s2 = (u + 2) % NBUF
      drain_gather(s)
      compute(c, s)
      issue_wb(c, s)
      if u < 2:
        @pl.when(p >= 1)
        def _():
          drain_wb(s2)
      else:
        drain_wb(s2)
      if u < 2:
        drain_idx(s2)
        fire(s2)
      else:
        @pl.when(c + 2 <= NCHUNK - 1)
        def _():
          drain_idx(s2)
          fire(s2)
      @pl.when(c + 4 <= NCHUNK - 1)
      def _():
        fire_idx(c + 4, s)
    return carry

  lax.fori_loop(0, NCHUNK // NBUF, body, 0)

  toff = pl.multiple_of(base + NCHUNK * CHUNK, 8)
  pltpu.sync_copy(nids32.at[pl.ds(toff, TAIL)], tixs)
  pltpu.sync_copy(nids32.at[pl.ds(N_EDGES + toff, TAIL)], tixt)
  pltpu.async_copy(table_sh.at[tixs], trows_s, g0)
  pltpu.async_copy(table_sh.at[tixt], trows_t, g1)
  pltpu.make_async_copy(table.at[pl.ds(0, TAIL)], trows_s, g0).wait()
  pltpu.make_async_copy(table.at[pl.ds(0, TAIL)], trows_t, g1).wait()
  dot_group(trows_s, trows_t, 0, NCHUNK * CHUNK)
  pltpu.async_copy(trows_s, srch_out.at[pl.ds(toff, TAIL)], g0)
  pltpu.async_copy(trows_t, tgth_out.at[pl.ds(toff, TAIL)], g1)

  drain_wb(2)
  drain_wb(3)
  pltpu.make_async_copy(trows_s, srch_out.at[pl.ds(0, TAIL)], g0).wait()
  pltpu.make_async_copy(trows_t, tgth_out.at[pl.ds(0, TAIL)], g1).wait()

  pltpu.sync_copy(score_all, score_out.at[pl.ds(base, EPW)])


@jax.jit
def kernel(node_embeddings, node_nids):
  nids32 = node_nids.astype(jnp.int32).reshape(-1)

  mesh = plsc.VectorSubcoreMesh(core_axis_name="c", subcore_axis_name="s")
  out_type = (
      jax.ShapeDtypeStruct((N_EDGES,), jnp.float32),
      jax.ShapeDtypeStruct((N_EDGES, D_FEAT), jnp.float32),
      jax.ShapeDtypeStruct((N_EDGES, D_FEAT), jnp.float32),
  )
  scratch = [
      pltpu.VMEM((EPW,), jnp.float32),
      pltpu.VMEM_SHARED((N_NODES, D_FEAT), jnp.float32),
  ] + [pltpu.VMEM((CHUNK, D_FEAT), jnp.float32) for _ in range(2 * NBUF)] + [
      pltpu.VMEM((CHUNK,), jnp.int32) for _ in range(2 * NBUF)
  ] + [
      pltpu.VMEM((TAIL, D_FEAT), jnp.float32),
      pltpu.VMEM((TAIL, D_FEAT), jnp.float32),
      pltpu.VMEM((TAIL,), jnp.int32),
      pltpu.VMEM((TAIL,), jnp.int32),
      pltpu.VMEM((2 * LANES,), jnp.float32),
  ] + [
      pltpu.SemaphoreType.DMA for _ in range(3 * NBUF)
  ]
  score, src_h, tgt_h = pl.kernel(
      _sc_body,
      out_type=out_type,
      mesh=mesh,
      scratch_types=scratch,
  )(node_embeddings, nids32)
  return (score, src_h, tgt_h)
